# baseline (speedup 1.0000x reference)
import functools

import jax
import jax.numpy as jnp
from jax import lax
from jax.experimental import pallas as pl
from jax.experimental.pallas import tpu as pltpu
from jax.experimental.pallas import tpu_sc as plsc

N = 10000
DF = 128
NCLS = 40
E = 320000

NC = 2
NS = 16
NW = NC * NS
K = 128
SUB = 80
CT = SUB * K
EP = NW * CT
NP = 10240
RPT = NP // NS
HALF = SUB // 2

_mesh = functools.partial(
    plsc.VectorSubcoreMesh,
    core_axis_name="c",
    subcore_axis_name="s",
    num_cores=NC,
    num_subcores=NS,
)


def _deg_body(dst_hbm, zeros_hbm, out_hbm, dst_v, ones_v, acc_sp):
    c = lax.axis_index("c")
    s = lax.axis_index("s")
    w = c * NS + s
    pltpu.sync_copy(dst_hbm.at[w], dst_v)
    ones16 = jnp.ones((16,), jnp.float32)
    for i in range(K // 16):
        ones_v[pl.ds(i * 16, 16)] = ones16
    pltpu.sync_copy(zeros_hbm.at[pl.ds(s * RPT, RPT)], acc_sp.at[pl.ds(s * RPT, RPT)])
    plsc.subcore_barrier()

    def body(j, carry):
        pltpu.sync_copy(ones_v, acc_sp.at[dst_v.at[j]], add=True)
        return carry

    lax.fori_loop(0, SUB, body, 0)
    plsc.subcore_barrier()
    pltpu.sync_copy(acc_sp.at[pl.ds(s * RPT, RPT)], out_hbm.at[c, pl.ds(s * RPT, RPT)])


def _edge_pass_body(table_hbm, src_hbm, dst_hbm, zeros_hbm, out_hbm,
                    src_v, dst_v, rows0, rows1, sem_a, sem_b, acc_sp):
    c = lax.axis_index("c")
    s = lax.axis_index("s")
    w = c * NS + s
    z16 = jnp.zeros((16,), jnp.int32)
    for i in range(K // 16):
        src_v[HALF, pl.ds(i * 16, 16)] = z16
    pltpu.sync_copy(zeros_hbm, acc_sp.at[pl.ds(s * RPT, RPT)])
    plsc.subcore_barrier()

    def gather(j, buf, sem):
        return pltpu.make_async_copy(table_hbm.at[src_v.at[j]], buf, sem)

    def body(t, carry):
        j = 2 * t
        gather(j + 1, rows1, sem_b).start()
        gather(j, rows0, sem_a).wait()
        pltpu.sync_copy(rows0, acc_sp.at[dst_v.at[j]], add=True)
        gather(j + 2, rows0, sem_a).start()
        gather(j + 1, rows1, sem_b).wait()
        pltpu.sync_copy(rows1, acc_sp.at[dst_v.at[j + 1]], add=True)
        return carry

    for h in range(SUB // HALF):
        pltpu.sync_copy(src_hbm.at[w, pl.ds(h * HALF, HALF)],
                        src_v.at[pl.ds(0, HALF)])
        pltpu.sync_copy(dst_hbm.at[w, pl.ds(h * HALF, HALF)], dst_v)
        gather(0, rows0, sem_a).start()
        lax.fori_loop(0, HALF // 2, body, 0)
        gather(HALF, rows0, sem_a).wait()
    plsc.subcore_barrier()
    pltpu.sync_copy(acc_sp.at[pl.ds(s * RPT, RPT)],
                    out_hbm.at[c, pl.ds(s * RPT, RPT)])


_deg_kernel = pl.kernel(
    _deg_body,
    out_type=jax.ShapeDtypeStruct((NC, NP), jnp.float32),
    mesh=_mesh(),
    scratch_types=[
        pltpu.VMEM((SUB, K), jnp.int32),
        pltpu.VMEM((K,), jnp.float32),
        pltpu.VMEM_SHARED((NP,), jnp.float32),
    ],
)

_edge_kernel = pl.kernel(
    _edge_pass_body,
    out_type=jax.ShapeDtypeStruct((NC, NP, DF), jnp.float32),
    mesh=_mesh(),
    scratch_types=[
        pltpu.VMEM((HALF + 1, K), jnp.int32),
        pltpu.VMEM((HALF, K), jnp.int32),
        pltpu.VMEM((K, DF), jnp.float32),
        pltpu.VMEM((K, DF), jnp.float32),
        pltpu.SemaphoreType.DMA,
        pltpu.SemaphoreType.DMA,
        pltpu.VMEM_SHARED((NP, DF), jnp.float32),
    ],
)


_R = 1000


def _tc1_body(x_ref, degp_ref, wpre_ref, bpre_ref, w1_ref, g1p_ref, dinv_ref):
    deg = degp_ref[0] + degp_ref[1] + 1.0
    dinv = lax.rsqrt(deg)
    h0 = jnp.dot(x_ref[...], wpre_ref[...], preferred_element_type=jnp.float32)
    h0 = h0 + bpre_ref[...]
    g1 = jnp.dot(h0, w1_ref[...], preferred_element_type=jnp.float32)
    g1p_ref[...] = g1 * dinv
    dinv_ref[...] = dinv


def _tc2_body(acc_ref, g1p_ref, dinv_ref, b1_ref, w2_ref, g2p_ref):
    dinv = dinv_ref[...]
    agg = dinv * (acc_ref[0] + acc_ref[1] + g1p_ref[...]) + b1_ref[...]
    h1 = jnp.maximum(agg, 0.0)
    g2p_ref[...] = jnp.dot(h1, w2_ref[...], preferred_element_type=jnp.float32) * dinv


def _tc3_body(acc_ref, g2p_ref, dinv_ref, b2_ref, wcls_ref, bcls_ref, out_ref):
    dinv = dinv_ref[...]
    h2 = dinv * (acc_ref[0] + acc_ref[1] + g2p_ref[...]) + b2_ref[...]
    nrm = jnp.sqrt(jnp.sum(h2 * h2, axis=-1, keepdims=True))
    h = h2 / jnp.maximum(nrm, 1e-12)
    logits = jnp.dot(h, wcls_ref[...], preferred_element_type=jnp.float32)
    logits = logits + bcls_ref[...]
    m = jnp.max(logits, axis=-1, keepdims=True)
    lse = m + jnp.log(jnp.sum(jnp.exp(logits - m), axis=-1, keepdims=True))
    out_ref[...] = logits - lse


def _row_spec(shape):
    if len(shape) == 2:
        return pl.BlockSpec((_R, shape[1]), lambda i: (i, 0))
    return pl.BlockSpec((shape[0], _R, shape[2]), lambda i: (0, i, 0))


def _full_spec(shape):
    nd = len(shape)
    return pl.BlockSpec(shape, lambda i: (0,) * nd)


def _tc1(x, degp, wpre, bpre, w1):
    return pl.pallas_call(
        _tc1_body,
        grid=(N // _R,),
        in_specs=[
            _row_spec((N, DF)),
            _row_spec((2, N, 1)),
            _full_spec((DF, DF)),
            _full_spec((1, DF)),
            _full_spec((DF, DF)),
        ],
        out_specs=[_row_spec((N, DF)), _row_spec((N, 1))],
        out_shape=[
            jax.ShapeDtypeStruct((N, DF), jnp.float32),
            jax.ShapeDtypeStruct((N, 1), jnp.float32),
        ],
    )(x, degp, wpre, bpre, w1)


def _tc2(acc, g1p, dinv, b1, w2):
    return pl.pallas_call(
        _tc2_body,
        grid=(N // _R,),
        in_specs=[
            _row_spec((2, N, DF)),
            _row_spec((N, DF)),
            _row_spec((N, 1)),
            _full_spec((1, DF)),
            _full_spec((DF, DF)),
        ],
        out_specs=_row_spec((N, DF)),
        out_shape=jax.ShapeDtypeStruct((N, DF), jnp.float32),
    )(acc, g1p, dinv, b1, w2)


def _tc3(acc, g2p, dinv, b2, wcls, bcls):
    return pl.pallas_call(
        _tc3_body,
        grid=(N // _R,),
        in_specs=[
            _row_spec((2, N, DF)),
            _row_spec((N, DF)),
            _row_spec((N, 1)),
            _full_spec((1, DF)),
            _full_spec((DF, NCLS)),
            _full_spec((1, NCLS)),
        ],
        out_specs=pl.BlockSpec((_R, NCLS), lambda i: (i, 0)),
        out_shape=jax.ShapeDtypeStruct((N, NCLS), jnp.float32),
    )(acc, g2p, dinv, b2, wcls, bcls)


def kernel(x, edge_index, W_pre, b_pre, W1, b1, W2, b2, W_cls, b_cls):
    src = edge_index[0]
    dst = edge_index[1]
    pad = EP - E
    srcp = jnp.concatenate([src, jnp.zeros((pad,), jnp.int32)]).reshape(NW, SUB, K)
    dstp = jnp.concatenate([dst, jnp.full((pad,), N, jnp.int32)]).reshape(NW, SUB, K)

    zflat = jnp.zeros((NP,), jnp.float32)
    zrows = jnp.zeros((RPT, DF), jnp.float32)

    degp = _deg_kernel(dstp, zflat)
    degp_sl = degp[:, :N, None]

    g1p, dinv = _tc1(x, degp_sl, W_pre, b_pre.reshape(1, DF), W1)

    acc1 = _edge_kernel(g1p, srcp, dstp, zrows)
    g2p = _tc2(acc1[:, :N, :], g1p, dinv, b1.reshape(1, DF), W2)

    acc2 = _edge_kernel(g2p, srcp, dstp, zrows)
    return _tc3(acc2[:, :N, :], g2p, dinv, b2.reshape(1, DF),
                W_cls, b_cls.reshape(1, NCLS))


# speedup vs baseline: 1.4661x; 1.4661x over previous
import functools

import jax
import jax.numpy as jnp
from jax import lax
from jax.experimental import pallas as pl
from jax.experimental.pallas import tpu as pltpu
from jax.experimental.pallas import tpu_sc as plsc

N = 10000
DF = 128
NCLS = 40
E = 320000

NC = 2
NS = 16
NW = NC * NS
K = 128
SUB = 80
CT = SUB * K
EP = NW * CT
NP = 10240
RPT = NP // NS
HALF = SUB // 2

_mesh = functools.partial(
    plsc.VectorSubcoreMesh,
    core_axis_name="c",
    subcore_axis_name="s",
    num_cores=NC,
    num_subcores=NS,
)


def _deg_body(dst_hbm, zeros_hbm, out_hbm, dst_v, ones_v, acc_sp):
    c = lax.axis_index("c")
    s = lax.axis_index("s")
    w = c * NS + s
    pltpu.sync_copy(dst_hbm.at[w], dst_v)
    ones16 = jnp.ones((16,), jnp.float32)
    for i in range(K // 16):
        ones_v[pl.ds(i * 16, 16)] = ones16
    pltpu.sync_copy(zeros_hbm.at[pl.ds(s * RPT, RPT)], acc_sp.at[pl.ds(s * RPT, RPT)])
    plsc.subcore_barrier()

    def body(j, carry):
        pltpu.sync_copy(ones_v, acc_sp.at[dst_v.at[j]], add=True)
        return carry

    lax.fori_loop(0, SUB, body, 0)
    plsc.subcore_barrier()
    pltpu.sync_copy(acc_sp.at[pl.ds(s * RPT, RPT)], out_hbm.at[c, pl.ds(s * RPT, RPT)])


def _edge_pass_body(table_hbm, src_hbm, dst_hbm, zeros_hbm, out_hbm,
                    src_v, dst_v, rows0, acc_sp):
    c = lax.axis_index("c")
    s = lax.axis_index("s")
    w = c * NS + s
    pltpu.sync_copy(src_hbm.at[w], src_v)
    pltpu.sync_copy(dst_hbm.at[w], dst_v)
    pltpu.sync_copy(zeros_hbm, acc_sp.at[pl.ds(s * RPT, RPT)])
    plsc.subcore_barrier()

    def body(j, carry):
        pltpu.sync_copy(table_hbm.at[src_v.at[j]], rows0)
        pltpu.sync_copy(rows0, acc_sp.at[dst_v.at[j]], add=True)
        return carry

    lax.fori_loop(0, SUB, body, 0)
    plsc.subcore_barrier()
    pltpu.sync_copy(acc_sp.at[pl.ds(s * RPT, RPT)],
                    out_hbm.at[c, pl.ds(s * RPT, RPT)])


_deg_kernel = pl.kernel(
    _deg_body,
    out_type=jax.ShapeDtypeStruct((NC, NP), jnp.float32),
    mesh=_mesh(),
    scratch_types=[
        pltpu.VMEM((SUB, K), jnp.int32),
        pltpu.VMEM((K,), jnp.float32),
        pltpu.VMEM_SHARED((NP,), jnp.float32),
    ],
)

_edge_kernel = pl.kernel(
    _edge_pass_body,
    out_type=jax.ShapeDtypeStruct((NC, NP, DF), jnp.float32),
    mesh=_mesh(),
    scratch_types=[
        pltpu.VMEM((SUB, K), jnp.int32),
        pltpu.VMEM((SUB, K), jnp.int32),
        pltpu.VMEM((K, DF), jnp.float32),
        pltpu.VMEM_SHARED((NP, DF), jnp.float32),
    ],
)


_R = 1000


def _tc1_body(x_ref, degp_ref, wpre_ref, bpre_ref, w1_ref, g1p_ref, dinv_ref):
    deg = degp_ref[0] + degp_ref[1] + 1.0
    dinv = lax.rsqrt(deg)
    h0 = jnp.dot(x_ref[...], wpre_ref[...], preferred_element_type=jnp.float32)
    h0 = h0 + bpre_ref[...]
    g1 = jnp.dot(h0, w1_ref[...], preferred_element_type=jnp.float32)
    g1p_ref[...] = g1 * dinv
    dinv_ref[...] = dinv


def _tc2_body(acc_ref, g1p_ref, dinv_ref, b1_ref, w2_ref, g2p_ref):
    dinv = dinv_ref[...]
    agg = dinv * (acc_ref[0] + acc_ref[1] + g1p_ref[...]) + b1_ref[...]
    h1 = jnp.maximum(agg, 0.0)
    g2p_ref[...] = jnp.dot(h1, w2_ref[...], preferred_element_type=jnp.float32) * dinv


def _tc3_body(acc_ref, g2p_ref, dinv_ref, b2_ref, wcls_ref, bcls_ref, out_ref):
    dinv = dinv_ref[...]
    h2 = dinv * (acc_ref[0] + acc_ref[1] + g2p_ref[...]) + b2_ref[...]
    nrm = jnp.sqrt(jnp.sum(h2 * h2, axis=-1, keepdims=True))
    h = h2 / jnp.maximum(nrm, 1e-12)
    logits = jnp.dot(h, wcls_ref[...], preferred_element_type=jnp.float32)
    logits = logits + bcls_ref[...]
    m = jnp.max(logits, axis=-1, keepdims=True)
    lse = m + jnp.log(jnp.sum(jnp.exp(logits - m), axis=-1, keepdims=True))
    out_ref[...] = logits - lse


def _row_spec(shape):
    if len(shape) == 2:
        return pl.BlockSpec((_R, shape[1]), lambda i: (i, 0))
    return pl.BlockSpec((shape[0], _R, shape[2]), lambda i: (0, i, 0))


def _full_spec(shape):
    nd = len(shape)
    return pl.BlockSpec(shape, lambda i: (0,) * nd)


def _tc1(x, degp, wpre, bpre, w1):
    return pl.pallas_call(
        _tc1_body,
        grid=(N // _R,),
        in_specs=[
            _row_spec((N, DF)),
            _row_spec((2, N, 1)),
            _full_spec((DF, DF)),
            _full_spec((1, DF)),
            _full_spec((DF, DF)),
        ],
        out_specs=[_row_spec((N, DF)), _row_spec((N, 1))],
        out_shape=[
            jax.ShapeDtypeStruct((N, DF), jnp.float32),
            jax.ShapeDtypeStruct((N, 1), jnp.float32),
        ],
    )(x, degp, wpre, bpre, w1)


def _tc2(acc, g1p, dinv, b1, w2):
    return pl.pallas_call(
        _tc2_body,
        grid=(N // _R,),
        in_specs=[
            _row_spec((2, N, DF)),
            _row_spec((N, DF)),
            _row_spec((N, 1)),
            _full_spec((1, DF)),
            _full_spec((DF, DF)),
        ],
        out_specs=_row_spec((N, DF)),
        out_shape=jax.ShapeDtypeStruct((N, DF), jnp.float32),
    )(acc, g1p, dinv, b1, w2)


def _tc3(acc, g2p, dinv, b2, wcls, bcls):
    return pl.pallas_call(
        _tc3_body,
        grid=(N // _R,),
        in_specs=[
            _row_spec((2, N, DF)),
            _row_spec((N, DF)),
            _row_spec((N, 1)),
            _full_spec((1, DF)),
            _full_spec((DF, NCLS)),
            _full_spec((1, NCLS)),
        ],
        out_specs=pl.BlockSpec((_R, NCLS), lambda i: (i, 0)),
        out_shape=jax.ShapeDtypeStruct((N, NCLS), jnp.float32),
    )(acc, g2p, dinv, b2, wcls, bcls)


def kernel(x, edge_index, W_pre, b_pre, W1, b1, W2, b2, W_cls, b_cls):
    src = edge_index[0]
    dst = edge_index[1]
    pad = EP - E
    srcp = jnp.concatenate([src, jnp.zeros((pad,), jnp.int32)]).reshape(NW, SUB, K)
    dstp = jnp.concatenate([dst, jnp.full((pad,), N, jnp.int32)]).reshape(NW, SUB, K)

    zflat = jnp.zeros((NP,), jnp.float32)
    zrows = jnp.zeros((RPT, DF), jnp.float32)

    degp = _deg_kernel(dstp, zflat)
    degp_sl = degp[:, :N, None]

    g1p, dinv = _tc1(x, degp_sl, W_pre, b_pre.reshape(1, DF), W1)

    acc1 = _edge_kernel(g1p, srcp, dstp, zrows)
    g2p = _tc2(acc1[:, :N, :], g1p, dinv, b1.reshape(1, DF), W2)

    acc2 = _edge_kernel(g2p, srcp, dstp, zrows)
    return _tc3(acc2[:, :N, :], g2p, dinv, b2.reshape(1, DF),
                W_cls, b_cls.reshape(1, NCLS))


# speedup vs baseline: 1.4686x; 1.0017x over previous
import functools

import jax
import jax.numpy as jnp
from jax import lax
from jax.experimental import pallas as pl
from jax.experimental.pallas import tpu as pltpu
from jax.experimental.pallas import tpu_sc as plsc

N = 10000
DF = 128
NCLS = 40
E = 320000

NC = 2
NS = 16
NW = NC * NS
K = 128
SUB = 80
CT = SUB * K
EP = NW * CT
NP = 10240
RPT = NP // NS
HALF = SUB // 2

_mesh = functools.partial(
    plsc.VectorSubcoreMesh,
    core_axis_name="c",
    subcore_axis_name="s",
    num_cores=NC,
    num_subcores=NS,
)


def _deg_body(dst_hbm, zeros_hbm, out_hbm, dst_v, ones_v, acc_sp):
    c = lax.axis_index("c")
    s = lax.axis_index("s")
    w = c * NS + s
    pltpu.sync_copy(dst_hbm.at[w], dst_v)
    ones16 = jnp.ones((16,), jnp.float32)
    for i in range(K // 16):
        ones_v[pl.ds(i * 16, 16)] = ones16
    pltpu.sync_copy(zeros_hbm.at[pl.ds(s * RPT, RPT)], acc_sp.at[pl.ds(s * RPT, RPT)])
    plsc.subcore_barrier()

    def body(j, carry):
        pltpu.sync_copy(ones_v, acc_sp.at[dst_v.at[j]], add=True)
        return carry

    lax.fori_loop(0, SUB, body, 0)
    plsc.subcore_barrier()
    pltpu.sync_copy(acc_sp.at[pl.ds(s * RPT, RPT)], out_hbm.at[c, pl.ds(s * RPT, RPT)])


def _edge_pass_body(table_hbm, src_hbm, dst_hbm, zeros_hbm, out_hbm,
                    src_v, dst_v, rows0, acc_sp):
    c = lax.axis_index("c")
    s = lax.axis_index("s")
    w = c * NS + s
    pltpu.sync_copy(src_hbm.at[w], src_v)
    pltpu.sync_copy(dst_hbm.at[w], dst_v)
    pltpu.sync_copy(zeros_hbm, acc_sp.at[pl.ds(s * RPT, RPT)])
    plsc.subcore_barrier()

    def body(j, carry):
        pltpu.sync_copy(table_hbm.at[src_v.at[j]], rows0)
        pltpu.sync_copy(rows0, acc_sp.at[dst_v.at[j]], add=True)
        return carry

    lax.fori_loop(0, SUB, body, 0)
    plsc.subcore_barrier()
    pltpu.sync_copy(acc_sp.at[pl.ds(s * RPT, RPT)],
                    out_hbm.at[c, pl.ds(s * RPT, RPT)])


_deg_kernel = pl.kernel(
    _deg_body,
    out_type=jax.ShapeDtypeStruct((NC, NP), jnp.float32),
    mesh=_mesh(),
    scratch_types=[
        pltpu.VMEM((SUB, K), jnp.int32),
        pltpu.VMEM((K,), jnp.float32),
        pltpu.VMEM_SHARED((NP,), jnp.float32),
    ],
)

_edge_kernel = pl.kernel(
    _edge_pass_body,
    out_type=jax.ShapeDtypeStruct((NC, NP, DF), jnp.float32),
    mesh=_mesh(),
    scratch_types=[
        pltpu.VMEM((SUB, K), jnp.int32),
        pltpu.VMEM((SUB, K), jnp.int32),
        pltpu.VMEM((K, DF), jnp.float32),
        pltpu.VMEM_SHARED((NP, DF), jnp.float32),
    ],
)


_R = 1000


def _tc1_body(x_ref, degp_ref, wpre_ref, bpre_ref, w1_ref, g1p_ref, dinv_ref):
    deg = degp_ref[0] + degp_ref[1] + 1.0
    dinv = lax.rsqrt(deg)
    h0 = jnp.dot(x_ref[...], wpre_ref[...], preferred_element_type=jnp.float32)
    h0 = h0 + bpre_ref[...]
    g1 = jnp.dot(h0, w1_ref[...], preferred_element_type=jnp.float32)
    g1p_ref[...] = g1 * dinv
    dinv_ref[...] = dinv


def _tc2_body(acc_ref, g1p_ref, dinv_ref, b1_ref, w2_ref, g2p_ref):
    dinv = dinv_ref[...]
    agg = dinv * (acc_ref[0] + acc_ref[1] + g1p_ref[...]) + b1_ref[...]
    h1 = jnp.maximum(agg, 0.0)
    g2p_ref[...] = jnp.dot(h1, w2_ref[...], preferred_element_type=jnp.float32) * dinv


def _tc3_body(acc_ref, g2p_ref, dinv_ref, b2_ref, wcls_ref, bcls_ref, out_ref):
    dinv = dinv_ref[...]
    h2 = dinv * (acc_ref[0] + acc_ref[1] + g2p_ref[...]) + b2_ref[...]
    nrm = jnp.sqrt(jnp.sum(h2 * h2, axis=-1, keepdims=True))
    h = h2 / jnp.maximum(nrm, 1e-12)
    logits = jnp.dot(h, wcls_ref[...], preferred_element_type=jnp.float32)
    logits = logits + bcls_ref[...]
    m = jnp.max(logits, axis=-1, keepdims=True)
    lse = m + jnp.log(jnp.sum(jnp.exp(logits - m), axis=-1, keepdims=True))
    out_ref[...] = logits - lse


def _row_spec(shape):
    if len(shape) == 2:
        return pl.BlockSpec((_R, shape[1]), lambda i: (i, 0))
    return pl.BlockSpec((shape[0], _R, shape[2]), lambda i: (0, i, 0))


def _full_spec(shape):
    nd = len(shape)
    return pl.BlockSpec(shape, lambda i: (0,) * nd)


def _tc1(x, degp, wpre, bpre, w1):
    return pl.pallas_call(
        _tc1_body,
        grid=(N // _R,),
        in_specs=[
            _row_spec((N, DF)),
            _row_spec((2, N, 1)),
            _full_spec((DF, DF)),
            _full_spec((1, DF)),
            _full_spec((DF, DF)),
        ],
        out_specs=[_row_spec((N, DF)), _row_spec((N, 1))],
        out_shape=[
            jax.ShapeDtypeStruct((N, DF), jnp.float32),
            jax.ShapeDtypeStruct((N, 1), jnp.float32),
        ],
    )(x, degp, wpre, bpre, w1)


def _tc2(acc, g1p, dinv, b1, w2):
    return pl.pallas_call(
        _tc2_body,
        grid=(N // _R,),
        in_specs=[
            _row_spec((2, N, DF)),
            _row_spec((N, DF)),
            _row_spec((N, 1)),
            _full_spec((1, DF)),
            _full_spec((DF, DF)),
        ],
        out_specs=_row_spec((N, DF)),
        out_shape=jax.ShapeDtypeStruct((N, DF), jnp.float32),
    )(acc, g1p, dinv, b1, w2)


def _tc3(acc, g2p, dinv, b2, wcls, bcls):
    return pl.pallas_call(
        _tc3_body,
        grid=(N // _R,),
        in_specs=[
            _row_spec((2, N, DF)),
            _row_spec((N, DF)),
            _row_spec((N, 1)),
            _full_spec((1, DF)),
            _full_spec((DF, NCLS)),
            _full_spec((1, NCLS)),
        ],
        out_specs=pl.BlockSpec((_R, NCLS), lambda i: (i, 0)),
        out_shape=jax.ShapeDtypeStruct((N, NCLS), jnp.float32),
    )(acc, g2p, dinv, b2, wcls, bcls)


def kernel(x, edge_index, W_pre, b_pre, W1, b1, W2, b2, W_cls, b_cls):
    src = edge_index[0]
    dst = edge_index[1]
    pad = EP - E
    srcp = jnp.concatenate([src, jnp.zeros((pad,), jnp.int32)]).reshape(NW, SUB, K)
    gar = N + jnp.arange(pad, dtype=jnp.int32) % (NP - N)
    dstp = jnp.concatenate([dst, gar]).reshape(NW, SUB, K)

    zflat = jnp.zeros((NP,), jnp.float32)
    zrows = jnp.zeros((RPT, DF), jnp.float32)

    degp = _deg_kernel(dstp, zflat)
    degp_sl = degp[:, :N, None]

    g1p, dinv = _tc1(x, degp_sl, W_pre, b_pre.reshape(1, DF), W1)

    acc1 = _edge_kernel(g1p, srcp, dstp, zrows)
    g2p = _tc2(acc1[:, :N, :], g1p, dinv, b1.reshape(1, DF), W2)

    acc2 = _edge_kernel(g2p, srcp, dstp, zrows)
    return _tc3(acc2[:, :N, :], g2p, dinv, b2.reshape(1, DF),
                W_cls, b_cls.reshape(1, NCLS))


# speedup vs baseline: 3.3165x; 2.2582x over previous
import functools

import jax
import jax.numpy as jnp
from jax import lax
from jax.experimental import pallas as pl
from jax.experimental.pallas import tpu as pltpu
from jax.experimental.pallas import tpu_sc as plsc

N = 10000
DF = 128
NCLS = 40
E = 320000

NC = 2
NS = 16
NW = NC * NS
K = 128
SUB = 79
CT = SUB * K
EP = NW * CT
NP = 10240
RPT = NP // NS
HALF = SUB // 2

_mesh = functools.partial(
    plsc.VectorSubcoreMesh,
    core_axis_name="c",
    subcore_axis_name="s",
    num_cores=NC,
    num_subcores=NS,
)


def _deg_body(dst_hbm, zeros_hbm, out_hbm, dst_v, ones_v, acc_sp):
    c = lax.axis_index("c")
    s = lax.axis_index("s")
    w = c * NS + s
    pltpu.sync_copy(dst_hbm.at[w], dst_v)
    ones16 = jnp.ones((16,), jnp.float32)
    for i in range(K // 16):
        ones_v[pl.ds(i * 16, 16)] = ones16
    pltpu.sync_copy(zeros_hbm.at[pl.ds(s * RPT, RPT)], acc_sp.at[pl.ds(s * RPT, RPT)])
    plsc.subcore_barrier()

    def body(j, carry):
        pltpu.sync_copy(ones_v, acc_sp.at[dst_v.at[j]], add=True)
        return carry

    lax.fori_loop(0, SUB, body, 0)
    plsc.subcore_barrier()
    pltpu.sync_copy(acc_sp.at[pl.ds(s * RPT, RPT)], out_hbm.at[c, pl.ds(s * RPT, RPT)])


def _edge_pass_body(table_hbm, src_hbm, dst_hbm, zeros_hbm, out_hbm,
                    src_v, dst_v, rows0, acc_sp):
    c = lax.axis_index("c")
    s = lax.axis_index("s")
    w = c * NS + s
    pltpu.sync_copy(src_hbm.at[w], src_v)
    pltpu.sync_copy(dst_hbm.at[w], dst_v)
    pltpu.sync_copy(zeros_hbm, acc_sp.at[pl.ds(s * RPT, RPT)])
    plsc.subcore_barrier()

    def body(j, carry):
        pltpu.sync_copy(table_hbm.at[src_v.at[j]], rows0)
        pltpu.sync_copy(rows0, acc_sp.at[dst_v.at[j]], add=True)
        return carry

    lax.fori_loop(0, SUB, body, 0)
    plsc.subcore_barrier()
    pltpu.sync_copy(acc_sp.at[pl.ds(s * RPT, RPT)],
                    out_hbm.at[c, pl.ds(s * RPT, RPT)])


_deg_kernel = pl.kernel(
    _deg_body,
    out_type=jax.ShapeDtypeStruct((NC, NP), jnp.float32),
    mesh=_mesh(),
    scratch_types=[
        pltpu.VMEM((SUB, K), jnp.int32),
        pltpu.VMEM((K,), jnp.float32),
        pltpu.VMEM_SHARED((NP,), jnp.float32),
    ],
)

_edge_kernel = pl.kernel(
    _edge_pass_body,
    out_type=jax.ShapeDtypeStruct((NC, NP, DF), jnp.float32),
    mesh=_mesh(),
    scratch_types=[
        pltpu.VMEM((SUB, K), jnp.int32),
        pltpu.VMEM((SUB, K), jnp.int32),
        pltpu.VMEM((K, DF), jnp.float32),
        pltpu.VMEM_SHARED((NP, DF), jnp.float32),
    ],
)


_R = 1000


def _tc1_body(x_ref, degp_ref, wpre_ref, bpre_ref, w1_ref, g1p_ref, dinv_ref):
    deg = degp_ref[0] + degp_ref[1] + 1.0
    dinv = lax.rsqrt(deg)
    h0 = jnp.dot(x_ref[...], wpre_ref[...], preferred_element_type=jnp.float32)
    h0 = h0 + bpre_ref[...]
    g1 = jnp.dot(h0, w1_ref[...], preferred_element_type=jnp.float32)
    g1p_ref[...] = g1 * dinv
    dinv_ref[...] = dinv


def _tc2_body(acc_ref, g1p_ref, dinv_ref, b1_ref, w2_ref, g2p_ref):
    dinv = dinv_ref[...]
    agg = dinv * (acc_ref[0] + acc_ref[1] + g1p_ref[...]) + b1_ref[...]
    h1 = jnp.maximum(agg, 0.0)
    g2p_ref[...] = jnp.dot(h1, w2_ref[...], preferred_element_type=jnp.float32) * dinv


def _tc3_body(acc_ref, g2p_ref, dinv_ref, b2_ref, wcls_ref, bcls_ref, out_ref):
    dinv = dinv_ref[...]
    h2 = dinv * (acc_ref[0] + acc_ref[1] + g2p_ref[...]) + b2_ref[...]
    nrm = jnp.sqrt(jnp.sum(h2 * h2, axis=-1, keepdims=True))
    h = h2 / jnp.maximum(nrm, 1e-12)
    logits = jnp.dot(h, wcls_ref[...], preferred_element_type=jnp.float32)
    logits = logits + bcls_ref[...]
    m = jnp.max(logits, axis=-1, keepdims=True)
    lse = m + jnp.log(jnp.sum(jnp.exp(logits - m), axis=-1, keepdims=True))
    out_ref[...] = logits - lse


def _row_spec(shape):
    if len(shape) == 2:
        return pl.BlockSpec((_R, shape[1]), lambda i: (i, 0))
    return pl.BlockSpec((shape[0], _R, shape[2]), lambda i: (0, i, 0))


def _full_spec(shape):
    nd = len(shape)
    return pl.BlockSpec(shape, lambda i: (0,) * nd)


def _tc1(x, degp, wpre, bpre, w1):
    return pl.pallas_call(
        _tc1_body,
        grid=(N // _R,),
        in_specs=[
            _row_spec((N, DF)),
            _row_spec((2, N, 1)),
            _full_spec((DF, DF)),
            _full_spec((1, DF)),
            _full_spec((DF, DF)),
        ],
        out_specs=[_row_spec((N, DF)), _row_spec((N, 1))],
        out_shape=[
            jax.ShapeDtypeStruct((N, DF), jnp.float32),
            jax.ShapeDtypeStruct((N, 1), jnp.float32),
        ],
    )(x, degp, wpre, bpre, w1)


def _tc2(acc, g1p, dinv, b1, w2):
    return pl.pallas_call(
        _tc2_body,
        grid=(N // _R,),
        in_specs=[
            _row_spec((2, N, DF)),
            _row_spec((N, DF)),
            _row_spec((N, 1)),
            _full_spec((1, DF)),
            _full_spec((DF, DF)),
        ],
        out_specs=_row_spec((N, DF)),
        out_shape=jax.ShapeDtypeStruct((N, DF), jnp.float32),
    )(acc, g1p, dinv, b1, w2)


def _tc3(acc, g2p, dinv, b2, wcls, bcls):
    return pl.pallas_call(
        _tc3_body,
        grid=(N // _R,),
        in_specs=[
            _row_spec((2, N, DF)),
            _row_spec((N, DF)),
            _row_spec((N, 1)),
            _full_spec((1, DF)),
            _full_spec((DF, NCLS)),
            _full_spec((1, NCLS)),
        ],
        out_specs=pl.BlockSpec((_R, NCLS), lambda i: (i, 0)),
        out_shape=jax.ShapeDtypeStruct((N, NCLS), jnp.float32),
    )(acc, g2p, dinv, b2, wcls, bcls)


def kernel(x, edge_index, W_pre, b_pre, W1, b1, W2, b2, W_cls, b_cls):
    src = edge_index[0]
    dst = edge_index[1]
    pad = EP - E
    ar = jnp.arange(pad, dtype=jnp.int32)
    srcp = jnp.concatenate([src, ar % N]).reshape(NW, SUB, K)
    gar = N + ar % (NP - N)
    dstp = jnp.concatenate([dst, gar]).reshape(NW, SUB, K)

    zflat = jnp.zeros((NP,), jnp.float32)
    zrows = jnp.zeros((RPT, DF), jnp.float32)

    degp = _deg_kernel(dstp, zflat)
    degp_sl = degp[:, :N, None]

    g1p, dinv = _tc1(x, degp_sl, W_pre, b_pre.reshape(1, DF), W1)

    acc1 = _edge_kernel(g1p, srcp, dstp, zrows)
    g2p = _tc2(acc1[:, :N, :], g1p, dinv, b1.reshape(1, DF), W2)

    acc2 = _edge_kernel(g2p, srcp, dstp, zrows)
    return _tc3(acc2[:, :N, :], g2p, dinv, b2.reshape(1, DF),
                W_cls, b_cls.reshape(1, NCLS))


# speedup vs baseline: 4.0772x; 1.2294x over previous
import functools

import jax
import jax.numpy as jnp
from jax import lax
from jax.experimental import pallas as pl
from jax.experimental.pallas import tpu as pltpu
from jax.experimental.pallas import tpu_sc as plsc

N = 10000
DF = 128
NCLS = 40
E = 320000

NC = 2
NS = 16
NW = NC * NS
K = 128
SUB = 80
HC = SUB // 2
CT = SUB * K
EP = NW * CT
NP = 10240
RPT = NP // NS
HALF = SUB // 2

_mesh = functools.partial(
    plsc.VectorSubcoreMesh,
    core_axis_name="c",
    subcore_axis_name="s",
    num_cores=NC,
    num_subcores=NS,
)


def _deg_body(dst_hbm, zeros_hbm, out_hbm, dst_v, ones_v, acc_sp):
    c = lax.axis_index("c")
    s = lax.axis_index("s")
    w = c * NS + s
    pltpu.sync_copy(dst_hbm.at[w], dst_v)
    ones16 = jnp.ones((16,), jnp.float32)
    for i in range(K // 16):
        ones_v[pl.ds(i * 16, 16)] = ones16
    pltpu.sync_copy(zeros_hbm.at[pl.ds(s * RPT, RPT)], acc_sp.at[pl.ds(s * RPT, RPT)])
    plsc.subcore_barrier()

    def body(j, carry):
        pltpu.sync_copy(ones_v, acc_sp.at[dst_v.at[j]], add=True)
        return carry

    lax.fori_loop(0, SUB, body, 0)
    plsc.subcore_barrier()
    pltpu.sync_copy(acc_sp.at[pl.ds(s * RPT, RPT)], out_hbm.at[c, pl.ds(s * RPT, RPT)])


def _edge_pass_body(table_hbm, src_hbm, dst_hbm, zeros_hbm, out_hbm,
                    src_v, dst_v, rows0, rows1, sem_a, sem_b, acc_sp):
    c = lax.axis_index("c")
    s = lax.axis_index("s")
    w = c * NS + s
    pltpu.sync_copy(zeros_hbm, acc_sp.at[pl.ds(s * RPT, RPT)])
    plsc.subcore_barrier()

    def gath(j, buf):
        pltpu.sync_copy(table_hbm.at[src_v.at[j]], buf)

    def scat_start(j, buf, sem):
        pltpu.async_copy(buf, acc_sp.at[dst_v.at[j]], sem, add=True)

    def scat_wait(buf, sem):
        pltpu.make_async_copy(buf, acc_sp.at[dst_v.at[0]], sem).wait()

    def body(t, carry):
        j = 2 * t
        scat_wait(rows0, sem_a)
        gath(j, rows0)
        scat_start(j, rows0, sem_a)
        scat_wait(rows1, sem_b)
        gath(j + 1, rows1)
        scat_start(j + 1, rows1, sem_b)
        return carry

    for h in range(SUB // HC):
        pltpu.sync_copy(src_hbm.at[w, pl.ds(h * HC, HC)], src_v)
        pltpu.sync_copy(dst_hbm.at[w, pl.ds(h * HC, HC)], dst_v)
        gath(0, rows0)
        scat_start(0, rows0, sem_a)
        gath(1, rows1)
        scat_start(1, rows1, sem_b)
        lax.fori_loop(1, HC // 2, body, 0)
        scat_wait(rows0, sem_a)
        scat_wait(rows1, sem_b)
    plsc.subcore_barrier()
    pltpu.sync_copy(acc_sp.at[pl.ds(s * RPT, RPT)],
                    out_hbm.at[c, pl.ds(s * RPT, RPT)])


_deg_kernel = pl.kernel(
    _deg_body,
    out_type=jax.ShapeDtypeStruct((NC, NP), jnp.float32),
    mesh=_mesh(),
    scratch_types=[
        pltpu.VMEM((SUB, K), jnp.int32),
        pltpu.VMEM((K,), jnp.float32),
        pltpu.VMEM_SHARED((NP,), jnp.float32),
    ],
)

_edge_kernel = pl.kernel(
    _edge_pass_body,
    out_type=jax.ShapeDtypeStruct((NC, NP, DF), jnp.float32),
    mesh=_mesh(),
    scratch_types=[
        pltpu.VMEM((HC, K), jnp.int32),
        pltpu.VMEM((HC, K), jnp.int32),
        pltpu.VMEM((K, DF), jnp.float32),
        pltpu.VMEM((K, DF), jnp.float32),
        pltpu.SemaphoreType.DMA,
        pltpu.SemaphoreType.DMA,
        pltpu.VMEM_SHARED((NP, DF), jnp.float32),
    ],
)


_R = 1000


def _tc1_body(x_ref, degp_ref, wpre_ref, bpre_ref, w1_ref, g1p_ref, dinv_ref):
    deg = degp_ref[0] + degp_ref[1] + 1.0
    dinv = lax.rsqrt(deg)
    h0 = jnp.dot(x_ref[...], wpre_ref[...], preferred_element_type=jnp.float32)
    h0 = h0 + bpre_ref[...]
    g1 = jnp.dot(h0, w1_ref[...], preferred_element_type=jnp.float32)
    g1p_ref[...] = g1 * dinv
    dinv_ref[...] = dinv


def _tc2_body(acc_ref, g1p_ref, dinv_ref, b1_ref, w2_ref, g2p_ref):
    dinv = dinv_ref[...]
    agg = dinv * (acc_ref[0] + acc_ref[1] + g1p_ref[...]) + b1_ref[...]
    h1 = jnp.maximum(agg, 0.0)
    g2p_ref[...] = jnp.dot(h1, w2_ref[...], preferred_element_type=jnp.float32) * dinv


def _tc3_body(acc_ref, g2p_ref, dinv_ref, b2_ref, wcls_ref, bcls_ref, out_ref):
    dinv = dinv_ref[...]
    h2 = dinv * (acc_ref[0] + acc_ref[1] + g2p_ref[...]) + b2_ref[...]
    nrm = jnp.sqrt(jnp.sum(h2 * h2, axis=-1, keepdims=True))
    h = h2 / jnp.maximum(nrm, 1e-12)
    logits = jnp.dot(h, wcls_ref[...], preferred_element_type=jnp.float32)
    logits = logits + bcls_ref[...]
    m = jnp.max(logits, axis=-1, keepdims=True)
    lse = m + jnp.log(jnp.sum(jnp.exp(logits - m), axis=-1, keepdims=True))
    out_ref[...] = logits - lse


def _row_spec(shape):
    if len(shape) == 2:
        return pl.BlockSpec((_R, shape[1]), lambda i: (i, 0))
    return pl.BlockSpec((shape[0], _R, shape[2]), lambda i: (0, i, 0))


def _full_spec(shape):
    nd = len(shape)
    return pl.BlockSpec(shape, lambda i: (0,) * nd)


def _tc1(x, degp, wpre, bpre, w1):
    return pl.pallas_call(
        _tc1_body,
        grid=(N // _R,),
        in_specs=[
            _row_spec((N, DF)),
            _row_spec((2, N, 1)),
            _full_spec((DF, DF)),
            _full_spec((1, DF)),
            _full_spec((DF, DF)),
        ],
        out_specs=[_row_spec((N, DF)), _row_spec((N, 1))],
        out_shape=[
            jax.ShapeDtypeStruct((N, DF), jnp.float32),
            jax.ShapeDtypeStruct((N, 1), jnp.float32),
        ],
    )(x, degp, wpre, bpre, w1)


def _tc2(acc, g1p, dinv, b1, w2):
    return pl.pallas_call(
        _tc2_body,
        grid=(N // _R,),
        in_specs=[
            _row_spec((2, N, DF)),
            _row_spec((N, DF)),
            _row_spec((N, 1)),
            _full_spec((1, DF)),
            _full_spec((DF, DF)),
        ],
        out_specs=_row_spec((N, DF)),
        out_shape=jax.ShapeDtypeStruct((N, DF), jnp.float32),
    )(acc, g1p, dinv, b1, w2)


def _tc3(acc, g2p, dinv, b2, wcls, bcls):
    return pl.pallas_call(
        _tc3_body,
        grid=(N // _R,),
        in_specs=[
            _row_spec((2, N, DF)),
            _row_spec((N, DF)),
            _row_spec((N, 1)),
            _full_spec((1, DF)),
            _full_spec((DF, NCLS)),
            _full_spec((1, NCLS)),
        ],
        out_specs=pl.BlockSpec((_R, NCLS), lambda i: (i, 0)),
        out_shape=jax.ShapeDtypeStruct((N, NCLS), jnp.float32),
    )(acc, g2p, dinv, b2, wcls, bcls)


def kernel(x, edge_index, W_pre, b_pre, W1, b1, W2, b2, W_cls, b_cls):
    src = edge_index[0]
    dst = edge_index[1]
    pad = EP - E
    ar = jnp.arange(pad, dtype=jnp.int32)
    srcp = jnp.concatenate([src, ar % N]).reshape(NW, SUB, K)
    gar = N + ar % (NP - N)
    dstp = jnp.concatenate([dst, gar]).reshape(NW, SUB, K)

    zflat = jnp.zeros((NP,), jnp.float32)
    zrows = jnp.zeros((RPT, DF), jnp.float32)

    degp = _deg_kernel(dstp, zflat)
    degp_sl = degp[:, :N, None]

    g1p, dinv = _tc1(x, degp_sl, W_pre, b_pre.reshape(1, DF), W1)

    acc1 = _edge_kernel(g1p, srcp, dstp, zrows)
    g2p = _tc2(acc1[:, :N, :], g1p, dinv, b1.reshape(1, DF), W2)

    acc2 = _edge_kernel(g2p, srcp, dstp, zrows)
    return _tc3(acc2[:, :N, :], g2p, dinv, b2.reshape(1, DF),
                W_cls, b_cls.reshape(1, NCLS))


# speedup vs baseline: 4.2497x; 1.0423x over previous
import functools

import jax
import jax.numpy as jnp
from jax import lax
from jax.experimental import pallas as pl
from jax.experimental.pallas import tpu as pltpu
from jax.experimental.pallas import tpu_sc as plsc

N = 10000
DF = 128
NCLS = 40
E = 320000

NC = 2
NS = 16
NW = NC * NS
K = 128
SUB = 80
HC = SUB // 2
CT = SUB * K
EP = NW * CT
NP = 10240
RPT = NP // NS
HALF = SUB // 2

_mesh = functools.partial(
    plsc.VectorSubcoreMesh,
    core_axis_name="c",
    subcore_axis_name="s",
    num_cores=NC,
    num_subcores=NS,
)


def _deg_body(dst_hbm, zeros_hbm, out_hbm, dst_v, ones_v, acc_sp):
    c = lax.axis_index("c")
    s = lax.axis_index("s")
    w = c * NS + s
    pltpu.sync_copy(dst_hbm.at[w], dst_v)
    ones16 = jnp.ones((16,), jnp.float32)
    for i in range(K // 16):
        ones_v[pl.ds(i * 16, 16)] = ones16
    pltpu.sync_copy(zeros_hbm.at[pl.ds(s * RPT, RPT)], acc_sp.at[pl.ds(s * RPT, RPT)])
    plsc.subcore_barrier()

    def body(j, carry):
        pltpu.sync_copy(ones_v, acc_sp.at[dst_v.at[j]], add=True)
        return carry

    lax.fori_loop(0, SUB, body, 0)
    plsc.subcore_barrier()
    pltpu.sync_copy(acc_sp.at[pl.ds(s * RPT, RPT)], out_hbm.at[c, pl.ds(s * RPT, RPT)])


def _edge_pass_body(table_hbm, src_hbm, dst_hbm, out_hbm,
                    src_v, dst_v, rows0, rows1, sem_a, sem_b, acc_sp):
    c = lax.axis_index("c")
    s = lax.axis_index("s")
    w = c * NS + s
    z16 = jnp.zeros((16,), jnp.float32)

    def zrow(r, carry):
        for i in range(DF // 16):
            rows1[r, pl.ds(i * 16, 16)] = z16
        return carry

    lax.fori_loop(0, K, zrow, 0)
    for kk in range(RPT // K):
        pltpu.sync_copy(rows1, acc_sp.at[pl.ds(s * RPT + kk * K, K)])
    plsc.subcore_barrier()

    def gath(j, buf):
        pltpu.sync_copy(table_hbm.at[src_v.at[j]], buf)

    def scat_start(j, buf, sem):
        pltpu.async_copy(buf, acc_sp.at[dst_v.at[j]], sem, add=True)

    def scat_wait(buf, sem):
        pltpu.make_async_copy(buf, acc_sp.at[dst_v.at[0]], sem).wait()

    def body(t, carry):
        j = 2 * t
        scat_wait(rows0, sem_a)
        gath(j, rows0)
        scat_start(j, rows0, sem_a)
        scat_wait(rows1, sem_b)
        gath(j + 1, rows1)
        scat_start(j + 1, rows1, sem_b)
        return carry

    for h in range(SUB // HC):
        pltpu.sync_copy(src_hbm.at[w, pl.ds(h * HC, HC)], src_v)
        pltpu.sync_copy(dst_hbm.at[w, pl.ds(h * HC, HC)], dst_v)
        gath(0, rows0)
        scat_start(0, rows0, sem_a)
        gath(1, rows1)
        scat_start(1, rows1, sem_b)
        lax.fori_loop(1, HC // 2, body, 0)
        scat_wait(rows0, sem_a)
        scat_wait(rows1, sem_b)
    plsc.subcore_barrier()
    pltpu.sync_copy(acc_sp.at[pl.ds(s * RPT, RPT)],
                    out_hbm.at[c, pl.ds(s * RPT, RPT)])


_deg_kernel = pl.kernel(
    _deg_body,
    out_type=jax.ShapeDtypeStruct((NC, NP), jnp.float32),
    mesh=_mesh(),
    scratch_types=[
        pltpu.VMEM((SUB, K), jnp.int32),
        pltpu.VMEM((K,), jnp.float32),
        pltpu.VMEM_SHARED((NP,), jnp.float32),
    ],
)

_edge_kernel = pl.kernel(
    _edge_pass_body,
    out_type=jax.ShapeDtypeStruct((NC, NP, DF), jnp.float32),
    name="edge_pass",
    mesh=_mesh(),
    scratch_types=[
        pltpu.VMEM((HC, K), jnp.int32),
        pltpu.VMEM((HC, K), jnp.int32),
        pltpu.VMEM((K, DF), jnp.float32),
        pltpu.VMEM((K, DF), jnp.float32),
        pltpu.SemaphoreType.DMA,
        pltpu.SemaphoreType.DMA,
        pltpu.VMEM_SHARED((NP, DF), jnp.float32),
    ],
)


_R = 2000


def _tca_body(x_ref, wpre_ref, bpre_ref, w1_ref, g1_ref):
    h0 = jnp.dot(x_ref[...], wpre_ref[...], preferred_element_type=jnp.float32)
    h0 = h0 + bpre_ref[...]
    g1_ref[...] = jnp.dot(h0, w1_ref[...], preferred_element_type=jnp.float32)


def _tcb_body(g1_ref, degp_ref, g1p_ref, dinv_ref):
    deg = degp_ref[0] + degp_ref[1] + 1.0
    dinv = lax.rsqrt(deg)
    g1p_ref[...] = g1_ref[...] * dinv
    dinv_ref[...] = dinv


def _tc2_body(acc_ref, g1p_ref, dinv_ref, b1_ref, w2_ref, g2p_ref):
    dinv = dinv_ref[...]
    agg = dinv * (acc_ref[0] + acc_ref[1] + g1p_ref[...]) + b1_ref[...]
    h1 = jnp.maximum(agg, 0.0)
    g2p_ref[...] = jnp.dot(h1, w2_ref[...], preferred_element_type=jnp.float32) * dinv


def _tc3_body(acc_ref, g2p_ref, dinv_ref, b2_ref, wcls_ref, bcls_ref, out_ref):
    dinv = dinv_ref[...]
    h2 = dinv * (acc_ref[0] + acc_ref[1] + g2p_ref[...]) + b2_ref[...]
    nrm = jnp.sqrt(jnp.sum(h2 * h2, axis=-1, keepdims=True))
    h = h2 / jnp.maximum(nrm, 1e-12)
    logits = jnp.dot(h, wcls_ref[...], preferred_element_type=jnp.float32)
    logits = logits + bcls_ref[...]
    m = jnp.max(logits, axis=-1, keepdims=True)
    lse = m + jnp.log(jnp.sum(jnp.exp(logits - m), axis=-1, keepdims=True))
    out_ref[...] = logits - lse


def _row_spec(shape):
    if len(shape) == 2:
        return pl.BlockSpec((_R, shape[1]), lambda i: (i, 0))
    return pl.BlockSpec((shape[0], _R, shape[2]), lambda i: (0, i, 0))


def _full_spec(shape):
    nd = len(shape)
    return pl.BlockSpec(shape, lambda i: (0,) * nd)


def _tca(x, wpre, bpre, w1):
    return pl.pallas_call(
        _tca_body,
        grid=(N // _R,),
        in_specs=[
            _row_spec((N, DF)),
            _full_spec((DF, DF)),
            _full_spec((1, DF)),
            _full_spec((DF, DF)),
        ],
        out_specs=_row_spec((N, DF)),
        out_shape=jax.ShapeDtypeStruct((N, DF), jnp.float32),
    )(x, wpre, bpre, w1)


def _tcb(g1, degp):
    return pl.pallas_call(
        _tcb_body,
        grid=(N // _R,),
        in_specs=[
            _row_spec((N, DF)),
            _row_spec((2, N, 1)),
        ],
        out_specs=[_row_spec((N, DF)), _row_spec((N, 1))],
        out_shape=[
            jax.ShapeDtypeStruct((N, DF), jnp.float32),
            jax.ShapeDtypeStruct((N, 1), jnp.float32),
        ],
    )(g1, degp)


def _tc2(acc, g1p, dinv, b1, w2):
    return pl.pallas_call(
        _tc2_body,
        grid=(N // _R,),
        in_specs=[
            _row_spec((2, N, DF)),
            _row_spec((N, DF)),
            _row_spec((N, 1)),
            _full_spec((1, DF)),
            _full_spec((DF, DF)),
        ],
        out_specs=_row_spec((N, DF)),
        out_shape=jax.ShapeDtypeStruct((N, DF), jnp.float32),
    )(acc, g1p, dinv, b1, w2)


def _tc3(acc, g2p, dinv, b2, wcls, bcls):
    return pl.pallas_call(
        _tc3_body,
        grid=(N // _R,),
        in_specs=[
            _row_spec((2, N, DF)),
            _row_spec((N, DF)),
            _row_spec((N, 1)),
            _full_spec((1, DF)),
            _full_spec((DF, NCLS)),
            _full_spec((1, NCLS)),
        ],
        out_specs=pl.BlockSpec((_R, NCLS), lambda i: (i, 0)),
        out_shape=jax.ShapeDtypeStruct((N, NCLS), jnp.float32),
    )(acc, g2p, dinv, b2, wcls, bcls)


def kernel(x, edge_index, W_pre, b_pre, W1, b1, W2, b2, W_cls, b_cls):
    src = edge_index[0]
    dst = edge_index[1]
    pad = EP - E
    ar = jnp.arange(pad, dtype=jnp.int32)
    srcp = jnp.concatenate([src, ar % N]).reshape(NW, SUB, K)
    gar = N + ar % (NP - N)
    dstp = jnp.concatenate([dst, gar]).reshape(NW, SUB, K)

    zflat = jnp.zeros((NP,), jnp.float32)

    degp = _deg_kernel(dstp, zflat)
    degp_sl = degp[:, :N, None]

    g1 = _tca(x, W_pre, b_pre.reshape(1, DF), W1)
    g1p, dinv = _tcb(g1, degp_sl)

    acc1 = _edge_kernel(g1p, srcp, dstp)
    g2p = _tc2(acc1[:, :N, :], g1p, dinv, b1.reshape(1, DF), W2)

    acc2 = _edge_kernel(g2p, srcp, dstp)
    return _tc3(acc2[:, :N, :], g2p, dinv, b2.reshape(1, DF),
                W_cls, b_cls.reshape(1, NCLS))


# speedup vs baseline: 4.5523x; 1.0712x over previous
import functools

import jax
import jax.numpy as jnp
from jax import lax
from jax.experimental import pallas as pl
from jax.experimental.pallas import tpu as pltpu
from jax.experimental.pallas import tpu_sc as plsc

N = 10000
DF = 128
NCLS = 40
E = 320000

NC = 2
NS = 16
NW = NC * NS
K = 128
SUB = 80
HC = SUB // 2
EC = E // K
TAIL = EC - (NW - 1) * SUB
NP = 10240
RPT = NP // NS

_mesh = functools.partial(
    plsc.VectorSubcoreMesh,
    core_axis_name="c",
    subcore_axis_name="s",
    num_cores=NC,
    num_subcores=NS,
)


def _deg_body(ei_hbm, zeros_hbm, out_hbm, dst_v, ones_v, sem_a, sem_b, acc_sp):
    c = lax.axis_index("c")
    s = lax.axis_index("s")
    w = c * NS + s
    last = w == NW - 1
    ones16 = jnp.ones((16,), jnp.float32)
    for i in range(K // 16):
        ones_v[pl.ds(i * 16, 16)] = ones16
    pltpu.sync_copy(zeros_hbm.at[pl.ds(s * RPT, RPT)], acc_sp.at[pl.ds(s * RPT, RPT)])
    plsc.subcore_barrier()

    def scat_start(j, sem):
        pltpu.async_copy(ones_v, acc_sp.at[dst_v.at[j]], sem, add=True)

    def scat_wait(sem):
        pltpu.make_async_copy(ones_v, acc_sp.at[dst_v.at[0]], sem).wait()

    def run(nch):
        scat_start(0, sem_a)
        scat_start(1, sem_b)

        def body(t, carry):
            scat_wait(sem_a)
            scat_start(2 * t, sem_a)
            scat_wait(sem_b)
            scat_start(2 * t + 1, sem_b)
            return carry

        lax.fori_loop(1, nch // 2, body, 0)
        scat_wait(sem_a)
        scat_wait(sem_b)

    for h in range(SUB // HC):
        @pl.when(jnp.logical_not(last))
        def _():
            pltpu.sync_copy(ei_hbm.at[1, pl.ds(w * SUB + h * HC, HC)], dst_v)
            run(HC)
        if h == 0:
            @pl.when(last)
            def _():
                pltpu.sync_copy(ei_hbm.at[1, pl.ds(EC - TAIL, TAIL)],
                                dst_v.at[pl.ds(0, TAIL)])
                run(TAIL)
    plsc.subcore_barrier()
    pltpu.sync_copy(acc_sp.at[pl.ds(s * RPT, RPT)], out_hbm.at[c, pl.ds(s * RPT, RPT)])


def _edge_pass_body(table_hbm, ei_hbm, out_hbm,
                    src_v, dst_v, rows0, rows1, sem_a, sem_b, acc_sp):
    c = lax.axis_index("c")
    s = lax.axis_index("s")
    w = c * NS + s
    last = w == NW - 1
    z16 = jnp.zeros((16,), jnp.float32)

    def zrow(r, carry):
        for i in range(DF // 16):
            rows1[r, pl.ds(i * 16, 16)] = z16
        return carry

    lax.fori_loop(0, K, zrow, 0)
    for kk in range(RPT // K):
        pltpu.sync_copy(rows1, acc_sp.at[pl.ds(s * RPT + kk * K, K)])
    plsc.subcore_barrier()

    def gath(j, buf):
        pltpu.sync_copy(table_hbm.at[src_v.at[j]], buf)

    def scat_start(j, buf, sem):
        pltpu.async_copy(buf, acc_sp.at[dst_v.at[j]], sem, add=True)

    def scat_wait(buf, sem):
        pltpu.make_async_copy(buf, acc_sp.at[dst_v.at[0]], sem).wait()

    def body(t, carry):
        j = 2 * t
        scat_wait(rows0, sem_a)
        gath(j, rows0)
        scat_start(j, rows0, sem_a)
        scat_wait(rows1, sem_b)
        gath(j + 1, rows1)
        scat_start(j + 1, rows1, sem_b)
        return carry

    def run(nch):
        gath(0, rows0)
        scat_start(0, rows0, sem_a)
        gath(1, rows1)
        scat_start(1, rows1, sem_b)
        lax.fori_loop(1, nch // 2, body, 0)
        scat_wait(rows0, sem_a)
        scat_wait(rows1, sem_b)

    for h in range(SUB // HC):
        @pl.when(jnp.logical_not(last))
        def _():
            pltpu.sync_copy(ei_hbm.at[0, pl.ds(w * SUB + h * HC, HC)], src_v)
            pltpu.sync_copy(ei_hbm.at[1, pl.ds(w * SUB + h * HC, HC)], dst_v)
            run(HC)
        if h == 0:
            @pl.when(last)
            def _():
                pltpu.sync_copy(ei_hbm.at[0, pl.ds(EC - TAIL, TAIL)],
                                src_v.at[pl.ds(0, TAIL)])
                pltpu.sync_copy(ei_hbm.at[1, pl.ds(EC - TAIL, TAIL)],
                                dst_v.at[pl.ds(0, TAIL)])
                run(TAIL)
    plsc.subcore_barrier()
    pltpu.sync_copy(acc_sp.at[pl.ds(s * RPT, RPT)],
                    out_hbm.at[c, pl.ds(s * RPT, RPT)])


_deg_kernel = pl.kernel(
    _deg_body,
    out_type=jax.ShapeDtypeStruct((NC, NP), jnp.float32),
    mesh=_mesh(),
    name="deg_pass",
    scratch_types=[
        pltpu.VMEM((HC, K), jnp.int32),
        pltpu.VMEM((K,), jnp.float32),
        pltpu.SemaphoreType.DMA,
        pltpu.SemaphoreType.DMA,
        pltpu.VMEM_SHARED((NP,), jnp.float32),
    ],
)

_edge_kernel = pl.kernel(
    _edge_pass_body,
    out_type=jax.ShapeDtypeStruct((NC, NP, DF), jnp.float32),
    name="edge_pass",
    mesh=_mesh(),
    scratch_types=[
        pltpu.VMEM((HC, K), jnp.int32),
        pltpu.VMEM((HC, K), jnp.int32),
        pltpu.VMEM((K, DF), jnp.float32),
        pltpu.VMEM((K, DF), jnp.float32),
        pltpu.SemaphoreType.DMA,
        pltpu.SemaphoreType.DMA,
        pltpu.VMEM_SHARED((NP, DF), jnp.float32),
    ],
)


_R = 2000


def _tca_body(x_ref, wpre_ref, bpre_ref, w1_ref, g1_ref):
    h0 = jnp.dot(x_ref[...], wpre_ref[...], preferred_element_type=jnp.float32)
    h0 = h0 + bpre_ref[...]
    g1_ref[...] = jnp.dot(h0, w1_ref[...], preferred_element_type=jnp.float32)


def _tcb_body(g1_ref, degp_ref, g1p_ref, dinv_ref):
    deg = degp_ref[0] + degp_ref[1] + 1.0
    dinv = lax.rsqrt(deg)
    g1p_ref[...] = g1_ref[...] * dinv
    dinv_ref[...] = dinv


def _tc2_body(acc_ref, g1p_ref, dinv_ref, b1_ref, w2_ref, g2p_ref):
    dinv = dinv_ref[...]
    agg = dinv * (acc_ref[0] + acc_ref[1] + g1p_ref[...]) + b1_ref[...]
    h1 = jnp.maximum(agg, 0.0)
    g2p_ref[...] = jnp.dot(h1, w2_ref[...], preferred_element_type=jnp.float32) * dinv


def _tc3_body(acc_ref, g2p_ref, dinv_ref, b2_ref, wcls_ref, bcls_ref, out_ref):
    dinv = dinv_ref[...]
    h2 = dinv * (acc_ref[0] + acc_ref[1] + g2p_ref[...]) + b2_ref[...]
    nrm = jnp.sqrt(jnp.sum(h2 * h2, axis=-1, keepdims=True))
    h = h2 / jnp.maximum(nrm, 1e-12)
    logits = jnp.dot(h, wcls_ref[...], preferred_element_type=jnp.float32)
    logits = logits + bcls_ref[...]
    m = jnp.max(logits, axis=-1, keepdims=True)
    lse = m + jnp.log(jnp.sum(jnp.exp(logits - m), axis=-1, keepdims=True))
    out_ref[...] = logits - lse


def _row_spec(shape):
    if len(shape) == 2:
        return pl.BlockSpec((_R, shape[1]), lambda i: (i, 0))
    return pl.BlockSpec((shape[0], _R, shape[2]), lambda i: (0, i, 0))


def _full_spec(shape):
    nd = len(shape)
    return pl.BlockSpec(shape, lambda i: (0,) * nd)


def _tca(x, wpre, bpre, w1):
    return pl.pallas_call(
        _tca_body,
        grid=(N // _R,),
        in_specs=[
            _row_spec((N, DF)),
            _full_spec((DF, DF)),
            _full_spec((1, DF)),
            _full_spec((DF, DF)),
        ],
        out_specs=_row_spec((N, DF)),
        out_shape=jax.ShapeDtypeStruct((N, DF), jnp.float32),
    )(x, wpre, bpre, w1)


def _tcb(g1, degp):
    return pl.pallas_call(
        _tcb_body,
        grid=(N // _R,),
        in_specs=[
            _row_spec((N, DF)),
            _row_spec((2, NP, 1)),
        ],
        out_specs=[_row_spec((N, DF)), _row_spec((N, 1))],
        out_shape=[
            jax.ShapeDtypeStruct((N, DF), jnp.float32),
            jax.ShapeDtypeStruct((N, 1), jnp.float32),
        ],
    )(g1, degp)


def _tc2(acc, g1p, dinv, b1, w2):
    return pl.pallas_call(
        _tc2_body,
        grid=(N // _R,),
        in_specs=[
            _row_spec((2, NP, DF)),
            _row_spec((N, DF)),
            _row_spec((N, 1)),
            _full_spec((1, DF)),
            _full_spec((DF, DF)),
        ],
        out_specs=_row_spec((N, DF)),
        out_shape=jax.ShapeDtypeStruct((N, DF), jnp.float32),
    )(acc, g1p, dinv, b1, w2)


def _tc3(acc, g2p, dinv, b2, wcls, bcls):
    return pl.pallas_call(
        _tc3_body,
        grid=(N // _R,),
        in_specs=[
            _row_spec((2, NP, DF)),
            _row_spec((N, DF)),
            _row_spec((N, 1)),
            _full_spec((1, DF)),
            _full_spec((DF, NCLS)),
            _full_spec((1, NCLS)),
        ],
        out_specs=pl.BlockSpec((_R, NCLS), lambda i: (i, 0)),
        out_shape=jax.ShapeDtypeStruct((N, NCLS), jnp.float32),
    )(acc, g2p, dinv, b2, wcls, bcls)


def kernel(x, edge_index, W_pre, b_pre, W1, b1, W2, b2, W_cls, b_cls):
    ei3 = edge_index.reshape(2, EC, K)
    zflat = jnp.zeros((NP,), jnp.float32)

    degp = _deg_kernel(ei3, zflat)
    degp3 = degp.reshape(NC, NP, 1)

    g1 = _tca(x, W_pre, b_pre.reshape(1, DF), W1)
    g1p, dinv = _tcb(g1, degp3)

    acc1 = _edge_kernel(g1p, ei3)
    g2p = _tc2(acc1, g1p, dinv, b1.reshape(1, DF), W2)

    acc2 = _edge_kernel(g2p, ei3)
    return _tc3(acc2, g2p, dinv, b2.reshape(1, DF),
                W_cls, b_cls.reshape(1, NCLS))


# speedup vs baseline: 4.6946x; 1.0312x over previous
import functools

import jax
import jax.numpy as jnp
from jax import lax
from jax.experimental import pallas as pl
from jax.experimental.pallas import tpu as pltpu
from jax.experimental.pallas import tpu_sc as plsc

N = 10000
DF = 128
NCLS = 40
E = 320000

NC = 2
NS = 16
NW = NC * NS
K = 128
SUB = 80
HC = SUB // 2
EC = E // K
TAIL = EC - (NW - 1) * SUB
NP = 10240
RPT = NP // NS

_mesh = functools.partial(
    plsc.VectorSubcoreMesh,
    core_axis_name="c",
    subcore_axis_name="s",
    num_cores=NC,
    num_subcores=NS,
)


def _deg_body(ei_hbm, zeros_hbm, out_hbm, dst_v, ones_v, sem_a, sem_b, acc_sp):
    c = lax.axis_index("c")
    s = lax.axis_index("s")
    w = c * NS + s
    last = w == NW - 1
    ones16 = jnp.ones((16,), jnp.float32)
    for i in range(K // 16):
        ones_v[pl.ds(i * 16, 16)] = ones16
    pltpu.sync_copy(zeros_hbm.at[pl.ds(s * RPT, RPT)], acc_sp.at[pl.ds(s * RPT, RPT)])
    plsc.subcore_barrier()

    def scat_start(j, sem):
        pltpu.async_copy(ones_v, acc_sp.at[dst_v.at[j]], sem, add=True)

    def scat_wait(sem):
        pltpu.make_async_copy(ones_v, acc_sp.at[dst_v.at[0]], sem).wait()

    def run(nch):
        scat_start(0, sem_a)
        scat_start(1, sem_b)

        def body(t, carry):
            scat_wait(sem_a)
            scat_start(2 * t, sem_a)
            scat_wait(sem_b)
            scat_start(2 * t + 1, sem_b)
            return carry

        lax.fori_loop(1, nch // 2, body, 0)
        scat_wait(sem_a)
        scat_wait(sem_b)

    for h in range(SUB // HC):
        @pl.when(jnp.logical_not(last))
        def _():
            pltpu.sync_copy(ei_hbm.at[1, pl.ds(w * SUB + h * HC, HC)], dst_v)
            run(HC)
        if h == 0:
            @pl.when(last)
            def _():
                pltpu.sync_copy(ei_hbm.at[1, pl.ds(EC - TAIL, TAIL)],
                                dst_v.at[pl.ds(0, TAIL)])
                run(TAIL)
    plsc.subcore_barrier()
    pltpu.sync_copy(acc_sp.at[pl.ds(s * RPT, RPT)], out_hbm.at[c, pl.ds(s * RPT, RPT)])


def _edge_pass_body(table_hbm, ei_hbm, out_hbm,
                    src_v, dst_v, rows0, rows1, sem_a, sem_b, acc_sp):
    c = lax.axis_index("c")
    s = lax.axis_index("s")
    w = c * NS + s
    last = w == NW - 1
    z16 = jnp.zeros((16,), jnp.float32)

    def zrow(r, carry):
        for i in range(DF // 16):
            rows1[r, pl.ds(i * 16, 16)] = z16
        return carry

    lax.fori_loop(0, K, zrow, 0)
    for kk in range(RPT // K):
        pltpu.sync_copy(rows1, acc_sp.at[pl.ds(s * RPT + kk * K, K)])
    plsc.subcore_barrier()

    def gath(j, buf):
        pltpu.sync_copy(table_hbm.at[src_v.at[j]], buf)

    def scat_start(j, buf, sem):
        pltpu.async_copy(buf, acc_sp.at[dst_v.at[j]], sem, add=True)

    def scat_wait(buf, sem):
        pltpu.make_async_copy(buf, acc_sp.at[dst_v.at[0]], sem).wait()

    def body(t, carry):
        j = 2 * t
        scat_wait(rows0, sem_a)
        gath(j, rows0)
        scat_start(j, rows0, sem_a)
        scat_wait(rows1, sem_b)
        gath(j + 1, rows1)
        scat_start(j + 1, rows1, sem_b)
        return carry

    def run(nch):
        gath(0, rows0)
        scat_start(0, rows0, sem_a)
        gath(1, rows1)
        scat_start(1, rows1, sem_b)
        lax.fori_loop(1, nch // 2, body, 0)
        scat_wait(rows0, sem_a)
        scat_wait(rows1, sem_b)

    for h in range(SUB // HC):
        @pl.when(jnp.logical_not(last))
        def _():
            pltpu.sync_copy(ei_hbm.at[0, pl.ds(w * SUB + h * HC, HC)], src_v)
            pltpu.sync_copy(ei_hbm.at[1, pl.ds(w * SUB + h * HC, HC)], dst_v)
            run(HC)
        if h == 0:
            @pl.when(last)
            def _():
                pltpu.sync_copy(ei_hbm.at[0, pl.ds(EC - TAIL, TAIL)],
                                src_v.at[pl.ds(0, TAIL)])
                pltpu.sync_copy(ei_hbm.at[1, pl.ds(EC - TAIL, TAIL)],
                                dst_v.at[pl.ds(0, TAIL)])
                run(TAIL)
    plsc.subcore_barrier()
    pltpu.sync_copy(acc_sp.at[pl.ds(s * RPT, RPT)],
                    out_hbm.at[c, pl.ds(s * RPT, RPT)])


_deg_kernel = pl.kernel(
    _deg_body,
    out_type=jax.ShapeDtypeStruct((NC, NP), jnp.float32),
    mesh=_mesh(),
    name="deg_pass",
    scratch_types=[
        pltpu.VMEM((HC, K), jnp.int32),
        pltpu.VMEM((K,), jnp.float32),
        pltpu.SemaphoreType.DMA,
        pltpu.SemaphoreType.DMA,
        pltpu.VMEM_SHARED((NP,), jnp.float32),
    ],
)

_edge_kernel = pl.kernel(
    _edge_pass_body,
    out_type=jax.ShapeDtypeStruct((NC, NP, DF), jnp.float32),
    name="edge_pass",
    mesh=_mesh(),
    scratch_types=[
        pltpu.VMEM((HC, K), jnp.int32),
        pltpu.VMEM((HC, K), jnp.int32),
        pltpu.VMEM((K, DF), jnp.float32),
        pltpu.VMEM((K, DF), jnp.float32),
        pltpu.SemaphoreType.DMA,
        pltpu.SemaphoreType.DMA,
        pltpu.VMEM_SHARED((NP, DF), jnp.float32),
    ],
)


_R = 2000


def _tca_body(x_ref, wpre_ref, bpre_ref, w1_ref, g1_ref):
    h0 = jnp.dot(x_ref[...], wpre_ref[...], preferred_element_type=jnp.float32)
    h0 = h0 + bpre_ref[...]
    g1_ref[...] = jnp.dot(h0, w1_ref[...], preferred_element_type=jnp.float32)


_R2 = NP // 5


def _tcb_body(g1_ref, degp_ref, g1p_ref, dinv_ref):
    deg = degp_ref[0] + degp_ref[1] + 1.0
    dinv = jnp.reshape(lax.rsqrt(deg), (_R2, 1))
    g1p_ref[...] = g1_ref[...] * dinv
    dinv_ref[...] = dinv


def _tc2_body(acc_ref, g1p_ref, dinv_ref, b1_ref, w2_ref, g2p_ref):
    dinv = dinv_ref[...]
    agg = dinv * (acc_ref[0] + acc_ref[1] + g1p_ref[...]) + b1_ref[...]
    h1 = jnp.maximum(agg, 0.0)
    g2p_ref[...] = jnp.dot(h1, w2_ref[...], preferred_element_type=jnp.float32) * dinv


def _tc3_body(acc_ref, g2p_ref, dinv_ref, b2_ref, wcls_ref, bcls_ref, out_ref):
    dinv = dinv_ref[...]
    h2 = dinv * (acc_ref[0] + acc_ref[1] + g2p_ref[...]) + b2_ref[...]
    nrm = jnp.sqrt(jnp.sum(h2 * h2, axis=-1, keepdims=True))
    h = h2 / jnp.maximum(nrm, 1e-12)
    logits = jnp.dot(h, wcls_ref[...], preferred_element_type=jnp.float32)
    logits = logits + bcls_ref[...]
    m = jnp.max(logits, axis=-1, keepdims=True)
    lse = m + jnp.log(jnp.sum(jnp.exp(logits - m), axis=-1, keepdims=True))
    out_ref[...] = logits - lse


def _row_spec(shape):
    if len(shape) == 2:
        return pl.BlockSpec((_R, shape[1]), lambda i: (i, 0))
    return pl.BlockSpec((shape[0], _R, shape[2]), lambda i: (0, i, 0))


def _full_spec(shape):
    nd = len(shape)
    return pl.BlockSpec(shape, lambda i: (0,) * nd)


def _tca(x, wpre, bpre, w1):
    return pl.pallas_call(
        _tca_body,
        grid=(N // _R,),
        in_specs=[
            _row_spec((N, DF)),
            _full_spec((DF, DF)),
            _full_spec((1, DF)),
            _full_spec((DF, DF)),
        ],
        out_specs=_row_spec((N, DF)),
        out_shape=jax.ShapeDtypeStruct((N, DF), jnp.float32),
    )(x, wpre, bpre, w1)


def _tcb(g1, degp):
    return pl.pallas_call(
        _tcb_body,
        grid=(NP // _R2,),
        in_specs=[
            pl.BlockSpec((_R2, DF), lambda i: (i, 0)),
            pl.BlockSpec((2, _R2), lambda i: (0, i)),
        ],
        out_specs=[
            pl.BlockSpec((_R2, DF), lambda i: (i, 0)),
            pl.BlockSpec((_R2, 1), lambda i: (i, 0)),
        ],
        out_shape=[
            jax.ShapeDtypeStruct((NP, DF), jnp.float32),
            jax.ShapeDtypeStruct((NP, 1), jnp.float32),
        ],
    )(g1, degp)


def _tc2(acc, g1p, dinv, b1, w2):
    return pl.pallas_call(
        _tc2_body,
        grid=(N // _R,),
        in_specs=[
            _row_spec((2, NP, DF)),
            _row_spec((N, DF)),
            _row_spec((N, 1)),
            _full_spec((1, DF)),
            _full_spec((DF, DF)),
        ],
        out_specs=_row_spec((N, DF)),
        out_shape=jax.ShapeDtypeStruct((N, DF), jnp.float32),
    )(acc, g1p, dinv, b1, w2)


def _tc3(acc, g2p, dinv, b2, wcls, bcls):
    return pl.pallas_call(
        _tc3_body,
        grid=(N // _R,),
        in_specs=[
            _row_spec((2, NP, DF)),
            _row_spec((N, DF)),
            _row_spec((N, 1)),
            _full_spec((1, DF)),
            _full_spec((DF, NCLS)),
            _full_spec((1, NCLS)),
        ],
        out_specs=pl.BlockSpec((_R, NCLS), lambda i: (i, 0)),
        out_shape=jax.ShapeDtypeStruct((N, NCLS), jnp.float32),
    )(acc, g2p, dinv, b2, wcls, bcls)


def kernel(x, edge_index, W_pre, b_pre, W1, b1, W2, b2, W_cls, b_cls):
    ei3 = edge_index.reshape(2, EC, K)
    zflat = jnp.zeros((NP,), jnp.float32)

    degp = _deg_kernel(ei3, zflat)

    g1 = _tca(x, W_pre, b_pre.reshape(1, DF), W1)
    g1p, dinv = _tcb(g1, degp)

    acc1 = _edge_kernel(g1p, ei3)
    g2p = _tc2(acc1, g1p, dinv, b1.reshape(1, DF), W2)

    acc2 = _edge_kernel(g2p, ei3)
    return _tc3(acc2, g2p, dinv, b2.reshape(1, DF),
                W_cls, b_cls.reshape(1, NCLS))


# speedup vs baseline: 4.7055x; 1.0023x over previous
import functools

import jax
import jax.numpy as jnp
from jax import lax
from jax.experimental import pallas as pl
from jax.experimental.pallas import tpu as pltpu
from jax.experimental.pallas import tpu_sc as plsc

N = 10000
DF = 128
NCLS = 40
E = 320000

NC = 2
NS = 16
NW = NC * NS
K = 128
SUB = 80
HC = SUB // 2
EC = E // K
TAIL = EC - (NW - 1) * SUB
NP = 10240
RPT = NP // NS

_mesh = functools.partial(
    plsc.VectorSubcoreMesh,
    core_axis_name="c",
    subcore_axis_name="s",
    num_cores=NC,
    num_subcores=NS,
)


def _deg_body(ei_hbm, zeros_hbm, out_hbm, dst_v, ones_v, sem_a, sem_b, acc_sp):
    c = lax.axis_index("c")
    s = lax.axis_index("s")
    w = c * NS + s
    last = w == NW - 1

    @pl.when(jnp.logical_not(last))
    def _():
        pltpu.sync_copy(ei_hbm.at[1, pl.ds(w * SUB, HC)], dst_v)

    @pl.when(last)
    def _():
        pltpu.sync_copy(ei_hbm.at[1, pl.ds(EC - TAIL, TAIL)],
                        dst_v.at[pl.ds(0, TAIL)])

    ones16 = jnp.ones((16,), jnp.float32)
    for i in range(K // 16):
        ones_v[pl.ds(i * 16, 16)] = ones16
    pltpu.sync_copy(zeros_hbm.at[pl.ds(s * RPT, RPT)], acc_sp.at[pl.ds(s * RPT, RPT)])
    plsc.subcore_barrier()

    def scat_start(j, sem):
        pltpu.async_copy(ones_v, acc_sp.at[dst_v.at[j]], sem, add=True)

    def scat_wait(sem):
        pltpu.make_async_copy(ones_v, acc_sp.at[dst_v.at[0]], sem).wait()

    DEEP = 6

    def run(nch):
        for j in range(DEEP):
            scat_start(j, sem_a)

        def body(t, carry):
            scat_wait(sem_a)
            scat_start(t, sem_a)
            return carry

        lax.fori_loop(DEEP, nch, body, 0)
        for _ in range(DEEP):
            scat_wait(sem_a)

    @pl.when(jnp.logical_not(last))
    def _():
        run(HC)
        pltpu.sync_copy(ei_hbm.at[1, pl.ds(w * SUB + HC, HC)], dst_v)
        run(HC)

    @pl.when(last)
    def _():
        run(TAIL)
    plsc.subcore_barrier()
    pltpu.sync_copy(acc_sp.at[pl.ds(s * RPT, RPT)], out_hbm.at[c, pl.ds(s * RPT, RPT)])


def _edge_pass_body(table_hbm, ei_hbm, out_hbm,
                    src_v, dst_v, rows0, rows1, sem_a, sem_b, acc_sp):
    c = lax.axis_index("c")
    s = lax.axis_index("s")
    w = c * NS + s
    last = w == NW - 1

    @pl.when(jnp.logical_not(last))
    def _():
        pltpu.sync_copy(ei_hbm.at[0, pl.ds(w * SUB, HC)], src_v)
        pltpu.sync_copy(ei_hbm.at[1, pl.ds(w * SUB, HC)], dst_v)

    @pl.when(last)
    def _():
        pltpu.sync_copy(ei_hbm.at[0, pl.ds(EC - TAIL, TAIL)],
                        src_v.at[pl.ds(0, TAIL)])
        pltpu.sync_copy(ei_hbm.at[1, pl.ds(EC - TAIL, TAIL)],
                        dst_v.at[pl.ds(0, TAIL)])

    z16 = jnp.zeros((16,), jnp.float32)

    def zrow(r, carry):
        for i in range(DF // 16):
            rows1[r, pl.ds(i * 16, 16)] = z16
        return carry

    lax.fori_loop(0, K, zrow, 0)
    for kk in range(RPT // K):
        pltpu.sync_copy(rows1, acc_sp.at[pl.ds(s * RPT + kk * K, K)])
    plsc.subcore_barrier()

    def gath(j, buf):
        pltpu.sync_copy(table_hbm.at[src_v.at[j]], buf)

    def scat_start(j, buf, sem):
        pltpu.async_copy(buf, acc_sp.at[dst_v.at[j]], sem, add=True)

    def scat_wait(buf, sem):
        pltpu.make_async_copy(buf, acc_sp.at[dst_v.at[0]], sem).wait()

    def body(t, carry):
        j = 2 * t
        scat_wait(rows0, sem_a)
        gath(j, rows0)
        scat_start(j, rows0, sem_a)
        scat_wait(rows1, sem_b)
        gath(j + 1, rows1)
        scat_start(j + 1, rows1, sem_b)
        return carry

    def run(nch):
        gath(0, rows0)
        scat_start(0, rows0, sem_a)
        gath(1, rows1)
        scat_start(1, rows1, sem_b)
        lax.fori_loop(1, nch // 2, body, 0)
        scat_wait(rows0, sem_a)
        scat_wait(rows1, sem_b)

    @pl.when(jnp.logical_not(last))
    def _():
        run(HC)
        pltpu.sync_copy(ei_hbm.at[0, pl.ds(w * SUB + HC, HC)], src_v)
        pltpu.sync_copy(ei_hbm.at[1, pl.ds(w * SUB + HC, HC)], dst_v)
        run(HC)

    @pl.when(last)
    def _():
        run(TAIL)
    plsc.subcore_barrier()
    pltpu.sync_copy(acc_sp.at[pl.ds(s * RPT, RPT)],
                    out_hbm.at[c, pl.ds(s * RPT, RPT)])


_deg_kernel = pl.kernel(
    _deg_body,
    out_type=jax.ShapeDtypeStruct((NC, NP), jnp.float32),
    mesh=_mesh(),
    name="deg_pass",
    scratch_types=[
        pltpu.VMEM((HC, K), jnp.int32),
        pltpu.VMEM((K,), jnp.float32),
        pltpu.SemaphoreType.DMA,
        pltpu.SemaphoreType.DMA,
        pltpu.VMEM_SHARED((NP,), jnp.float32),
    ],
)

_edge_kernel = pl.kernel(
    _edge_pass_body,
    out_type=jax.ShapeDtypeStruct((NC, NP, DF), jnp.float32),
    name="edge_pass",
    mesh=_mesh(),
    scratch_types=[
        pltpu.VMEM((HC, K), jnp.int32),
        pltpu.VMEM((HC, K), jnp.int32),
        pltpu.VMEM((K, DF), jnp.float32),
        pltpu.VMEM((K, DF), jnp.float32),
        pltpu.SemaphoreType.DMA,
        pltpu.SemaphoreType.DMA,
        pltpu.VMEM_SHARED((NP, DF), jnp.float32),
    ],
)


_R = 2000


def _tca_body(x_ref, wpre_ref, bpre_ref, w1_ref, g1_ref):
    h0 = jnp.dot(x_ref[...], wpre_ref[...], preferred_element_type=jnp.float32)
    h0 = h0 + bpre_ref[...]
    g1_ref[...] = jnp.dot(h0, w1_ref[...], preferred_element_type=jnp.float32)


_R2 = NP // 5


def _tcb_body(g1_ref, degp_ref, g1p_ref, dinv_ref):
    deg = degp_ref[0] + degp_ref[1] + 1.0
    dinv = jnp.reshape(lax.rsqrt(deg), (_R2, 1))
    g1p_ref[...] = g1_ref[...] * dinv
    dinv_ref[...] = dinv


def _tc2_body(acc_ref, g1p_ref, dinv_ref, b1_ref, w2_ref, g2p_ref):
    dinv = dinv_ref[...]
    agg = dinv * (acc_ref[0] + acc_ref[1] + g1p_ref[...]) + b1_ref[...]
    h1 = jnp.maximum(agg, 0.0)
    g2p_ref[...] = jnp.dot(h1, w2_ref[...], preferred_element_type=jnp.float32) * dinv


def _tc3_body(acc_ref, g2p_ref, dinv_ref, b2_ref, wcls_ref, bcls_ref, out_ref):
    dinv = dinv_ref[...]
    h2 = dinv * (acc_ref[0] + acc_ref[1] + g2p_ref[...]) + b2_ref[...]
    nrm = jnp.sqrt(jnp.sum(h2 * h2, axis=-1, keepdims=True))
    h = h2 / jnp.maximum(nrm, 1e-12)
    logits = jnp.dot(h, wcls_ref[...], preferred_element_type=jnp.float32)
    logits = logits + bcls_ref[...]
    m = jnp.max(logits, axis=-1, keepdims=True)
    lse = m + jnp.log(jnp.sum(jnp.exp(logits - m), axis=-1, keepdims=True))
    out_ref[...] = logits - lse


def _row_spec(shape):
    if len(shape) == 2:
        return pl.BlockSpec((_R, shape[1]), lambda i: (i, 0))
    return pl.BlockSpec((shape[0], _R, shape[2]), lambda i: (0, i, 0))


def _full_spec(shape):
    nd = len(shape)
    return pl.BlockSpec(shape, lambda i: (0,) * nd)


def _tca(x, wpre, bpre, w1):
    return pl.pallas_call(
        _tca_body,
        grid=(N // _R,),
        in_specs=[
            _row_spec((N, DF)),
            _full_spec((DF, DF)),
            _full_spec((1, DF)),
            _full_spec((DF, DF)),
        ],
        out_specs=_row_spec((N, DF)),
        out_shape=jax.ShapeDtypeStruct((N, DF), jnp.float32),
    )(x, wpre, bpre, w1)


def _tcb(g1, degp):
    return pl.pallas_call(
        _tcb_body,
        grid=(NP // _R2,),
        in_specs=[
            pl.BlockSpec((_R2, DF), lambda i: (i, 0)),
            pl.BlockSpec((2, _R2), lambda i: (0, i)),
        ],
        out_specs=[
            pl.BlockSpec((_R2, DF), lambda i: (i, 0)),
            pl.BlockSpec((_R2, 1), lambda i: (i, 0)),
        ],
        out_shape=[
            jax.ShapeDtypeStruct((NP, DF), jnp.float32),
            jax.ShapeDtypeStruct((NP, 1), jnp.float32),
        ],
    )(g1, degp)


def _tc2(acc, g1p, dinv, b1, w2):
    return pl.pallas_call(
        _tc2_body,
        grid=(N // _R,),
        in_specs=[
            _row_spec((2, NP, DF)),
            _row_spec((N, DF)),
            _row_spec((N, 1)),
            _full_spec((1, DF)),
            _full_spec((DF, DF)),
        ],
        out_specs=_row_spec((N, DF)),
        out_shape=jax.ShapeDtypeStruct((N, DF), jnp.float32),
    )(acc, g1p, dinv, b1, w2)


def _tc3(acc, g2p, dinv, b2, wcls, bcls):
    return pl.pallas_call(
        _tc3_body,
        grid=(N // _R,),
        in_specs=[
            _row_spec((2, NP, DF)),
            _row_spec((N, DF)),
            _row_spec((N, 1)),
            _full_spec((1, DF)),
            _full_spec((DF, NCLS)),
            _full_spec((1, NCLS)),
        ],
        out_specs=pl.BlockSpec((_R, NCLS), lambda i: (i, 0)),
        out_shape=jax.ShapeDtypeStruct((N, NCLS), jnp.float32),
    )(acc, g2p, dinv, b2, wcls, bcls)


def kernel(x, edge_index, W_pre, b_pre, W1, b1, W2, b2, W_cls, b_cls):
    ei3 = edge_index.reshape(2, EC, K)
    zflat = jnp.zeros((NP,), jnp.float32)

    degp = _deg_kernel(ei3, zflat)

    g1 = _tca(x, W_pre, b_pre.reshape(1, DF), W1)
    g1p, dinv = _tcb(g1, degp)

    acc1 = _edge_kernel(g1p, ei3)
    g2p = _tc2(acc1, g1p, dinv, b1.reshape(1, DF), W2)

    acc2 = _edge_kernel(g2p, ei3)
    return _tc3(acc2, g2p, dinv, b2.reshape(1, DF),
                W_cls, b_cls.reshape(1, NCLS))
